# Initial kernel scaffold; baseline (speedup 1.0000x reference)
import jax
import jax.numpy as jnp
from jax.experimental import pallas as pl


def kernel(bottom_data, bottom_rois):
    raise NotImplementedError("write your pallas kernel here")


# speedup vs baseline: 2.8079x; 2.8079x over previous
import functools

import jax
import jax.numpy as jnp
from jax import lax
from jax.experimental import pallas as pl
from jax.experimental.pallas import tpu as pltpu
from jax.experimental.pallas import tpu_sc as plsc

_SCALE = 0.0625
_RS = 7
_SR = 2
_PD = 10
_NBINS = _RS * _RS
_H = 64
_W = 64
_NIMG = 4
_R = 512
_L = 16
_NW = 32
_RPW = _R // _NW
_OROWS = _RPW * _NBINS * _L // 128


def _sc_call(table, rois_flat):
  mesh = plsc.VectorSubcoreMesh(core_axis_name="c", subcore_axis_name="s")

  @functools.partial(
      pl.kernel,
      out_type=jax.ShapeDtypeStruct((_NW, _OROWS, 128), jnp.float32),
      mesh=mesh,
      scratch_types=[
          pltpu.VMEM((5 * _L,), jnp.float32),
          pltpu.VMEM((2 * _RS, _L), jnp.int32),
          pltpu.VMEM((2 * _RS, _L), jnp.float32),
          pltpu.VMEM((2 * _RS, _L), jnp.float32),
          pltpu.VMEM((2 * _RS, _L), jnp.int32),
          pltpu.VMEM((2 * _RS, _L), jnp.float32),
          pltpu.VMEM((2 * _RS, _L), jnp.float32),
          pltpu.VMEM((2 * _NBINS, 128), jnp.int32),
          pltpu.VMEM((2, 16 * _RPW, _L), jnp.float32),
          pltpu.VMEM((_OROWS, 128), jnp.float32),
          pltpu.SemaphoreType.DMA,
          pltpu.SemaphoreType.DMA,
      ],
      compiler_params=pltpu.CompilerParams(use_tc_tiling_on_sc=False),
  )
  def k(t_hbm, rois_hbm, out_hbm, rois_v, y0b, wy0b, wy1b, x0b, wx0b, wx1b,
        idxb, rowsb, outv, sem0, sem1):
    wid = lax.axis_index("s") * 2 + lax.axis_index("c")
    pltpu.sync_copy(rois_hbm.at[pl.ds(wid * (5 * _L), 5 * _L)], rois_v)

    rowbase = rois_v[pl.ds(0, _L)].astype(jnp.int32) * (_NBINS * _H * _W)
    sw = rois_v[pl.ds(1 * _L, _L)] * _SCALE
    sh = rois_v[pl.ds(2 * _L, _L)] * _SCALE
    ew = rois_v[pl.ds(3 * _L, _L)] * _SCALE
    eh = rois_v[pl.ds(4 * _L, _L)] * _SCALE
    bin_w = jnp.maximum(ew - sw, 0.1) / float(_RS)
    bin_h = jnp.maximum(eh - sh, 0.1) / float(_RS)

    def prep(start, binsz, size, lob, w0b, w1b):
      for p in range(_RS):
        for i in range(_SR):
          g = (i + 0.5) / _SR
          t = start + (p + g) * binsz
          mf = jnp.where((t >= -1.0) & (t <= float(size)), 1.0, 0.0)
          tc = jnp.clip(t, 0.0, float(size - 1))
          lo = jnp.minimum(tc.astype(jnp.int32), size - 2)
          fr = tc - lo.astype(jnp.float32)
          lob[p * _SR + i] = lo
          w0b[p * _SR + i] = (1.0 - fr) * mf
          w1b[p * _SR + i] = fr * mf

    prep(sh, bin_h, _H, y0b, wy0b, wy1b)
    prep(sw, bin_w, _W, x0b, wx0b, wx1b)

    samples = ((0, 0), (0, 1), (1, 0), (1, 1))

    def build(bin_, carry):
      ph = bin_ // _RS
      pw = bin_ - ph * _RS
      rb = rowbase + bin_ * (_H * _W)
      y0a = y0b[ph * 2]
      y0c = y0b[ph * 2 + 1]
      x0a = x0b[pw * 2]
      x0c = x0b[pw * 2 + 1]
      yrow = ((y0a * _W, (y0a + 1) * _W), (y0c * _W, (y0c + 1) * _W))
      xcol = ((x0a, x0a + 1), (x0c, x0c + 1))
      for s, (iy, ix) in enumerate(samples):
        for c, (cy, cx) in enumerate(samples):
          sc = s * 4 + c
          idx = rb + yrow[iy][cy] + xcol[ix][cx]
          idxb[2 * bin_ + sc // 8, pl.ds((sc % 8) * _L, _L)] = idx
      return carry

    lax.fori_loop(0, _NBINS, build, 0)

    sems = (sem0, sem1)

    def copies(bin_, slot):
      c0 = pltpu.make_async_copy(
          t_hbm.at[idxb.at[2 * bin_]],
          rowsb.at[slot, pl.ds(0, 8 * _L)], sems[slot])
      c1 = pltpu.make_async_copy(
          t_hbm.at[idxb.at[2 * bin_ + 1]],
          rowsb.at[slot, pl.ds(8 * _L, 8 * _L)], sems[slot])
      return c0, c1

    def compute(bin_, slot):
      ph = bin_ // _RS
      pw = bin_ - ph * _RS
      ry = ph * 2
      rx = pw * 2
      wyr = ((wy0b[ry], wy1b[ry]), (wy0b[ry + 1], wy1b[ry + 1]))
      wxr = ((wx0b[rx], wx1b[rx]), (wx0b[rx + 1], wx1b[rx + 1]))
      for i in range(_RPW):
        v = None
        for s, (iy, ix) in enumerate(samples):
          g0 = rowsb[slot, (s * 4 + 0) * _L + i]
          g1 = rowsb[slot, (s * 4 + 1) * _L + i]
          g2 = rowsb[slot, (s * 4 + 2) * _L + i]
          g3 = rowsb[slot, (s * 4 + 3) * _L + i]
          a = wxr[ix][0][i] * g0 + wxr[ix][1][i] * g1
          b = wxr[ix][0][i] * g2 + wxr[ix][1][i] * g3
          vs = wyr[iy][0][i] * a + wyr[iy][1][i] * b
          v = vs if v is None else jnp.maximum(v, vs)
        outv[bin_ * 2 + i // 8, pl.ds((i % 8) * _L, _L)] = v

    first0, first1 = copies(0, 0)
    first0.start()
    first1.start()

    def pair(p, carry):
      for par in range(2):
        bin_ = 2 * p + par
        nxt = bin_ + 1

        @pl.when(nxt < _NBINS)
        def _():
          n0, n1 = copies(nxt, 1 - par)
          n0.start()
          n1.start()

        @pl.when(bin_ < _NBINS)
        def _():
          w0, w1 = copies(bin_, par)
          w0.wait()
          w1.wait()
          compute(bin_, par)
      return carry

    lax.fori_loop(0, (_NBINS + 1) // 2, pair, 0)
    pltpu.sync_copy(outv, out_hbm.at[wid])

  return k(table, rois_flat)


def kernel(bottom_data, bottom_rois):
  t = bottom_data.reshape(_NIMG, _PD, _NBINS, _H, _W).transpose(0, 2, 3, 4, 1)
  t = jnp.pad(t, ((0, 0), (0, 0), (0, 0), (0, 0), (0, _L - _PD)))
  t = t.reshape(_NIMG * _NBINS * _H * _W, _L)
  rois_flat = (bottom_rois.reshape(_NW, _RPW, 5)
               .transpose(0, 2, 1).reshape(_NW * 5 * _RPW))
  raw = _sc_call(t, rois_flat)
  out = (raw.reshape(_NW, _NBINS, _RPW, _L)
         .transpose(0, 2, 3, 1)[:, :, :_PD, :]
         .reshape(_R, _PD, _RS, _RS))
  return out


# speedup vs baseline: 7.3732x; 2.6259x over previous
import functools

import jax
import jax.numpy as jnp
from jax import lax
from jax.experimental import pallas as pl
from jax.experimental.pallas import tpu as pltpu
from jax.experimental.pallas import tpu_sc as plsc

_SCALE = 0.0625
_RS = 7
_SR = 2
_PD = 10
_NBINS = _RS * _RS
_H = 64
_W = 64
_NIMG = 4
_R = 512
_L = 16
_NW = 32
_RPW = _R // _NW
_TROWS = _NIMG * _NBINS * _H * _W
_NPAIR = _NIMG * _NBINS
_NTASK = 2 * _NPAIR
_YH = _H // 2


def _sc_relayout(bottom_data):
  mesh = plsc.VectorSubcoreMesh(core_axis_name="c", subcore_axis_name="s")

  @functools.partial(
      pl.kernel,
      out_type=jax.ShapeDtypeStruct((_TROWS, _L), jnp.float32),
      mesh=mesh,
      scratch_types=[
          pltpu.VMEM((_PD, _YH, _W), jnp.float32),
          pltpu.VMEM((_YH * _W, _L), jnp.float32),
          pltpu.SemaphoreType.DMA,
      ],
      compiler_params=pltpu.CompilerParams(use_tc_tiling_on_sc=False, needs_layout_passes=False),
  )
  def k(bd_hbm, tab_hbm, inb, outb, sem):
    wid = lax.axis_index("s") * 2 + lax.axis_index("c")
    iotav = lax.iota(jnp.int32, _L)

    def task_body(task):
      pair = task // 2
      yh = task - pair * 2
      y0 = yh * _YH
      b = pair // _NBINS
      bin_ = pair - b * _NBINS
      for d in range(_PD):
        pltpu.async_copy(
            bd_hbm.at[b, d * _NBINS + bin_, pl.ds(y0, _YH)],
            inb.at[d], sem)
      for d in range(_PD):
        pltpu.make_async_copy(
            bd_hbm.at[b, d * _NBINS + bin_, pl.ds(y0, _YH)],
            inb.at[d], sem).wait()

      def row(yy, carry):
        rbase = yy * _W
        for d in range(_PD):
          cidx = jnp.full((_L,), d, jnp.int32)
          for xb in range(_W // _L):
            v = inb[d, yy, pl.ds(xb * _L, _L)]
            ridx = rbase + xb * _L + iotav
            plsc.store_scatter(outb, (ridx, cidx), v)
        return carry

      lax.fori_loop(0, _YH, row, 0)
      pltpu.sync_copy(
          outb, tab_hbm.at[pl.ds(pair * (_H * _W) + yh * (_YH * _W),
                                 _YH * _W)])

    nfull = _NTASK // _NW
    for kk in range(nfull):
      task_body(kk * _NW + wid)
    rest = _NTASK - nfull * _NW

    @pl.when(wid < rest)
    def _():
      task_body(nfull * _NW + wid)

  return k(bottom_data)


def _sc_gather(table, rois_flat):
  mesh = plsc.VectorSubcoreMesh(core_axis_name="c", subcore_axis_name="s")

  @functools.partial(
      pl.kernel,
      out_type=jax.ShapeDtypeStruct((_R * _PD * _NBINS,), jnp.float32),
      mesh=mesh,
      scratch_types=[
          pltpu.VMEM((5 * _L,), jnp.float32),
          pltpu.VMEM((2 * _RS, _L), jnp.int32),
          pltpu.VMEM((2 * _RS, _L), jnp.float32),
          pltpu.VMEM((2 * _RS, _L), jnp.float32),
          pltpu.VMEM((2 * _RS, _L), jnp.int32),
          pltpu.VMEM((2 * _RS, _L), jnp.float32),
          pltpu.VMEM((2 * _RS, _L), jnp.float32),
          pltpu.VMEM((2 * _NBINS, 128), jnp.int32),
          pltpu.VMEM((2, 16 * _RPW, _L), jnp.float32),
          pltpu.VMEM((_RPW * _PD * _NBINS,), jnp.float32),
          pltpu.SemaphoreType.DMA,
          pltpu.SemaphoreType.DMA,
      ],
      compiler_params=pltpu.CompilerParams(use_tc_tiling_on_sc=False, needs_layout_passes=False),
  )
  def k(t_hbm, rois_hbm, out_hbm, rois_v, y0b, wy0b, wy1b, x0b, wx0b, wx1b,
        idxb, rowsb, outw, sem0, sem1):
    wid = lax.axis_index("s") * 2 + lax.axis_index("c")
    pltpu.sync_copy(rois_hbm.at[pl.ds(wid * (5 * _L), 5 * _L)], rois_v)
    iotav = lax.iota(jnp.int32, _L)
    dmask = iotav < _PD

    rowbase = rois_v[pl.ds(0, _L)].astype(jnp.int32) * (_NBINS * _H * _W)
    sw = rois_v[pl.ds(1 * _L, _L)] * _SCALE
    sh = rois_v[pl.ds(2 * _L, _L)] * _SCALE
    ew = rois_v[pl.ds(3 * _L, _L)] * _SCALE
    eh = rois_v[pl.ds(4 * _L, _L)] * _SCALE
    bin_w = jnp.maximum(ew - sw, 0.1) / float(_RS)
    bin_h = jnp.maximum(eh - sh, 0.1) / float(_RS)

    def prep(start, binsz, size, lob, w0b, w1b):
      for p in range(_RS):
        for i in range(_SR):
          g = (i + 0.5) / _SR
          t = start + (p + g) * binsz
          mf = jnp.where((t >= -1.0) & (t <= float(size)), 1.0, 0.0)
          tc = jnp.clip(t, 0.0, float(size - 1))
          lo = jnp.minimum(tc.astype(jnp.int32), size - 2)
          fr = tc - lo.astype(jnp.float32)
          lob[p * _SR + i] = lo
          w0b[p * _SR + i] = (1.0 - fr) * mf
          w1b[p * _SR + i] = fr * mf

    prep(sh, bin_h, _H, y0b, wy0b, wy1b)
    prep(sw, bin_w, _W, x0b, wx0b, wx1b)

    samples = ((0, 0), (0, 1), (1, 0), (1, 1))

    def build(bin_, carry):
      ph = bin_ // _RS
      pw = bin_ - ph * _RS
      rb = rowbase + bin_ * (_H * _W)
      y0a = y0b[ph * 2]
      y0c = y0b[ph * 2 + 1]
      x0a = x0b[pw * 2]
      x0c = x0b[pw * 2 + 1]
      yrow = ((y0a * _W, (y0a + 1) * _W), (y0c * _W, (y0c + 1) * _W))
      xcol = ((x0a, x0a + 1), (x0c, x0c + 1))
      for s, (iy, ix) in enumerate(samples):
        for c, (cy, cx) in enumerate(samples):
          sc = s * 4 + c
          idx = rb + yrow[iy][cy] + xcol[ix][cx]
          idxb[2 * bin_ + sc // 8, pl.ds((sc % 8) * _L, _L)] = idx
      return carry

    lax.fori_loop(0, _NBINS, build, 0)

    sems = (sem0, sem1)

    def copies(bin_, slot):
      c0 = pltpu.make_async_copy(
          t_hbm.at[idxb.at[2 * bin_]],
          rowsb.at[slot, pl.ds(0, 8 * _L)], sems[slot])
      c1 = pltpu.make_async_copy(
          t_hbm.at[idxb.at[2 * bin_ + 1]],
          rowsb.at[slot, pl.ds(8 * _L, 8 * _L)], sems[slot])
      return c0, c1

    def compute(bin_, slot):
      ph = bin_ // _RS
      pw = bin_ - ph * _RS
      ry = ph * 2
      rx = pw * 2
      wyr = ((wy0b[ry], wy1b[ry]), (wy0b[ry + 1], wy1b[ry + 1]))
      wxr = ((wx0b[rx], wx1b[rx]), (wx0b[rx + 1], wx1b[rx + 1]))
      obase = iotav * _NBINS + bin_
      for i in range(_RPW):
        v = None
        for s, (iy, ix) in enumerate(samples):
          g0 = rowsb[slot, (s * 4 + 0) * _L + i]
          g1 = rowsb[slot, (s * 4 + 1) * _L + i]
          g2 = rowsb[slot, (s * 4 + 2) * _L + i]
          g3 = rowsb[slot, (s * 4 + 3) * _L + i]
          a = wxr[ix][0][i] * g0 + wxr[ix][1][i] * g1
          b = wxr[ix][0][i] * g2 + wxr[ix][1][i] * g3
          vs = wyr[iy][0][i] * a + wyr[iy][1][i] * b
          v = vs if v is None else jnp.maximum(v, vs)
        plsc.store_scatter(outw, (obase + i * (_PD * _NBINS),), v, mask=dmask)

    first0, first1 = copies(0, 0)
    first0.start()
    first1.start()

    def pair(p, carry):
      for par in range(2):
        bin_ = 2 * p + par
        nxt = bin_ + 1

        @pl.when(nxt < _NBINS)
        def _():
          n0, n1 = copies(nxt, 1 - par)
          n0.start()
          n1.start()

        @pl.when(bin_ < _NBINS)
        def _():
          w0, w1 = copies(bin_, par)
          w0.wait()
          w1.wait()
          compute(bin_, par)
      return carry

    lax.fori_loop(0, (_NBINS + 1) // 2, pair, 0)
    pltpu.sync_copy(
        outw, out_hbm.at[pl.ds(wid * (_RPW * _PD * _NBINS),
                               _RPW * _PD * _NBINS)])

  return k(table, rois_flat)


def kernel(bottom_data, bottom_rois):
  table = _sc_relayout(bottom_data)
  rois_flat = (bottom_rois.reshape(_NW, _RPW, 5)
               .transpose(0, 2, 1).reshape(_NW * 5 * _RPW))
  out = _sc_gather(table, rois_flat)
  return out.reshape(_R, _PD, _RS, _RS)


# speedup vs baseline: 8.5062x; 1.1537x over previous
import functools

import jax
import jax.numpy as jnp
from jax import lax
from jax.experimental import pallas as pl
from jax.experimental.pallas import tpu as pltpu
from jax.experimental.pallas import tpu_sc as plsc

_SCALE = 0.0625
_RS = 7
_SR = 2
_PD = 10
_NBINS = _RS * _RS
_H = 64
_W = 64
_NIMG = 4
_R = 512
_L = 16
_NW = 32
_RPW = _R // _NW
_TROWS = _NIMG * _NBINS * _H * _W
_NPAIR = _NIMG * _NBINS
_NTASK = 2 * _NPAIR
_YH = _H // 2


def _sc_relayout(bd_flat):
  mesh = plsc.VectorSubcoreMesh(core_axis_name="c", subcore_axis_name="s")
  chunk = _YH * _W

  @functools.partial(
      pl.kernel,
      out_type=jax.ShapeDtypeStruct((_TROWS, _L), jnp.float32),
      mesh=mesh,
      scratch_types=[
          pltpu.VMEM((2, _PD, chunk), jnp.float32),
          pltpu.VMEM((2, chunk, _L), jnp.float32),
          pltpu.SemaphoreType.DMA,
          pltpu.SemaphoreType.DMA,
          pltpu.SemaphoreType.DMA,
          pltpu.SemaphoreType.DMA,
      ],
      compiler_params=pltpu.CompilerParams(use_tc_tiling_on_sc=False, needs_layout_passes=False),
  )
  def k(bd_hbm, tab_hbm, inb, outb, si0, si1, so0, so1):
    wid = lax.axis_index("s") * 2 + lax.axis_index("c")
    iotav = lax.iota(jnp.int32, _L)
    sin = (si0, si1)
    sout = (so0, so1)

    def in_copies(task, slot):
      pair = task // 2
      yh = task - pair * 2
      b = pair // _NBINS
      bin_ = pair - b * _NBINS
      cs = []
      for d in range(_PD):
        off = ((b * (_PD * _NBINS) + d * _NBINS + bin_) * _H + yh * _YH) * _W
        cs.append(pltpu.make_async_copy(
            bd_hbm.at[pl.ds(off, chunk)], inb.at[slot, d], sin[slot]))
      return cs

    def out_copy(task, slot):
      return pltpu.make_async_copy(
          outb.at[slot], tab_hbm.at[pl.ds(task * chunk, chunk)], sout[slot])

    def scatter(slot):
      def row(yy, carry):
        rbase = yy * _W + iotav
        for d in range(_PD):
          cidx = jnp.full((_L,), d, jnp.int32)
          for xb in range(_W // _L):
            v = inb[slot, d, pl.ds(yy * _W + xb * _L, _L)]
            plsc.store_scatter(outb.at[slot], (rbase + xb * _L, cidx), v)
        return carry

      lax.fori_loop(0, _YH, row, 0)

    nfull = _NTASK // _NW
    rest = _NTASK - nfull * _NW

    def task_of(kk):
      return kk * _NW + wid

    for c in in_copies(task_of(0), 0):
      c.start()
    for kk in range(nfull + 1):
      par = kk % 2
      last = kk == nfull
      if not last:
        if kk + 1 < nfull:
          for c in in_copies(task_of(kk + 1), 1 - par):
            c.start()
        else:
          @pl.when(wid < rest)
          def _():
            for c in in_copies(task_of(kk + 1), 1 - par):
              c.start()
        for c in in_copies(task_of(kk), par):
          c.wait()
        if kk >= 2:
          out_copy(task_of(kk - 2), par).wait()
        scatter(par)
        out_copy(task_of(kk), par).start()
      else:
        @pl.when(wid < rest)
        def _():
          for c in in_copies(task_of(kk), par):
            c.wait()
          out_copy(task_of(kk - 2), par).wait()
          scatter(par)
          out_copy(task_of(kk), par).start()

    out_copy(task_of(nfull - 1), (nfull - 1) % 2).wait()

    @pl.when(wid < rest)
    def _():
      out_copy(task_of(nfull), nfull % 2).wait()

    @pl.when(wid >= rest)
    def _():
      out_copy(task_of(nfull - 2), nfull % 2).wait()

  return k(bd_flat)


def _sc_gather(table, rois_flat):
  mesh = plsc.VectorSubcoreMesh(core_axis_name="c", subcore_axis_name="s")

  @functools.partial(
      pl.kernel,
      out_type=jax.ShapeDtypeStruct((_R * _PD * _NBINS,), jnp.float32),
      mesh=mesh,
      scratch_types=[
          pltpu.VMEM((5 * _L,), jnp.float32),
          pltpu.VMEM((2 * _RS, _L), jnp.int32),
          pltpu.VMEM((2 * _RS, _L), jnp.float32),
          pltpu.VMEM((2 * _RS, _L), jnp.float32),
          pltpu.VMEM((2 * _RS, _L), jnp.int32),
          pltpu.VMEM((2 * _RS, _L), jnp.float32),
          pltpu.VMEM((2 * _RS, _L), jnp.float32),
          pltpu.VMEM((2 * _NBINS, 128), jnp.int32),
          pltpu.VMEM((2, 16 * _RPW, _L), jnp.float32),
          pltpu.VMEM((_RPW * _PD * _NBINS,), jnp.float32),
          pltpu.SemaphoreType.DMA,
          pltpu.SemaphoreType.DMA,
      ],
      compiler_params=pltpu.CompilerParams(use_tc_tiling_on_sc=False, needs_layout_passes=False),
  )
  def k(t_hbm, rois_hbm, out_hbm, rois_v, y0b, wy0b, wy1b, x0b, wx0b, wx1b,
        idxb, rowsb, outw, sem0, sem1):
    wid = lax.axis_index("s") * 2 + lax.axis_index("c")
    pltpu.sync_copy(rois_hbm.at[pl.ds(wid * (5 * _L), 5 * _L)], rois_v)
    iotav = lax.iota(jnp.int32, _L)
    dmask = iotav < _PD

    rowbase = rois_v[pl.ds(0, _L)].astype(jnp.int32) * (_NBINS * _H * _W)
    sw = rois_v[pl.ds(1 * _L, _L)] * _SCALE
    sh = rois_v[pl.ds(2 * _L, _L)] * _SCALE
    ew = rois_v[pl.ds(3 * _L, _L)] * _SCALE
    eh = rois_v[pl.ds(4 * _L, _L)] * _SCALE
    bin_w = jnp.maximum(ew - sw, 0.1) / float(_RS)
    bin_h = jnp.maximum(eh - sh, 0.1) / float(_RS)

    def prep(start, binsz, size, lob, w0b, w1b):
      for p in range(_RS):
        for i in range(_SR):
          g = (i + 0.5) / _SR
          t = start + (p + g) * binsz
          mf = jnp.where((t >= -1.0) & (t <= float(size)), 1.0, 0.0)
          tc = jnp.clip(t, 0.0, float(size - 1))
          lo = jnp.minimum(tc.astype(jnp.int32), size - 2)
          fr = tc - lo.astype(jnp.float32)
          lob[p * _SR + i] = lo
          w0b[p * _SR + i] = (1.0 - fr) * mf
          w1b[p * _SR + i] = fr * mf

    prep(sh, bin_h, _H, y0b, wy0b, wy1b)
    prep(sw, bin_w, _W, x0b, wx0b, wx1b)

    samples = ((0, 0), (0, 1), (1, 0), (1, 1))

    def build(bin_, carry):
      ph = bin_ // _RS
      pw = bin_ - ph * _RS
      rb = rowbase + bin_ * (_H * _W)
      y0a = y0b[ph * 2]
      y0c = y0b[ph * 2 + 1]
      x0a = x0b[pw * 2]
      x0c = x0b[pw * 2 + 1]
      yrow = ((y0a * _W, (y0a + 1) * _W), (y0c * _W, (y0c + 1) * _W))
      xcol = ((x0a, x0a + 1), (x0c, x0c + 1))
      for s, (iy, ix) in enumerate(samples):
        for c, (cy, cx) in enumerate(samples):
          sc = s * 4 + c
          idx = rb + yrow[iy][cy] + xcol[ix][cx]
          idxb[2 * bin_ + sc // 8, pl.ds((sc % 8) * _L, _L)] = idx
      return carry

    lax.fori_loop(0, _NBINS, build, 0)

    sems = (sem0, sem1)

    def copies(bin_, slot):
      c0 = pltpu.make_async_copy(
          t_hbm.at[idxb.at[2 * bin_]],
          rowsb.at[slot, pl.ds(0, 8 * _L)], sems[slot])
      c1 = pltpu.make_async_copy(
          t_hbm.at[idxb.at[2 * bin_ + 1]],
          rowsb.at[slot, pl.ds(8 * _L, 8 * _L)], sems[slot])
      return c0, c1

    def compute(bin_, slot):
      ph = bin_ // _RS
      pw = bin_ - ph * _RS
      ry = ph * 2
      rx = pw * 2
      wyr = ((wy0b[ry], wy1b[ry]), (wy0b[ry + 1], wy1b[ry + 1]))
      wxr = ((wx0b[rx], wx1b[rx]), (wx0b[rx + 1], wx1b[rx + 1]))
      obase = iotav * _NBINS + bin_
      for i in range(_RPW):
        v = None
        for s, (iy, ix) in enumerate(samples):
          g0 = rowsb[slot, (s * 4 + 0) * _L + i]
          g1 = rowsb[slot, (s * 4 + 1) * _L + i]
          g2 = rowsb[slot, (s * 4 + 2) * _L + i]
          g3 = rowsb[slot, (s * 4 + 3) * _L + i]
          a = wxr[ix][0][i] * g0 + wxr[ix][1][i] * g1
          b = wxr[ix][0][i] * g2 + wxr[ix][1][i] * g3
          vs = wyr[iy][0][i] * a + wyr[iy][1][i] * b
          v = vs if v is None else jnp.maximum(v, vs)
        plsc.store_scatter(outw, (obase + i * (_PD * _NBINS),), v, mask=dmask)

    first0, first1 = copies(0, 0)
    first0.start()
    first1.start()

    def pair(p, carry):
      for par in range(2):
        bin_ = 2 * p + par
        nxt = bin_ + 1

        @pl.when(nxt < _NBINS)
        def _():
          n0, n1 = copies(nxt, 1 - par)
          n0.start()
          n1.start()

        @pl.when(bin_ < _NBINS)
        def _():
          w0, w1 = copies(bin_, par)
          w0.wait()
          w1.wait()
          compute(bin_, par)
      return carry

    lax.fori_loop(0, (_NBINS + 1) // 2, pair, 0)
    pltpu.sync_copy(
        outw, out_hbm.at[pl.ds(wid * (_RPW * _PD * _NBINS),
                               _RPW * _PD * _NBINS)])

  return k(table, rois_flat)


def kernel(bottom_data, bottom_rois):
  table = _sc_relayout(bottom_data.reshape(_NIMG * _PD * _NBINS * _H * _W))
  rois_flat = (bottom_rois.reshape(_NW, _RPW, 5)
               .transpose(0, 2, 1).reshape(_NW * 5 * _RPW))
  out = _sc_gather(table, rois_flat)
  return out.reshape(_R, _PD, _RS, _RS)


# speedup vs baseline: 8.6035x; 1.0114x over previous
import functools

import jax
import jax.numpy as jnp
from jax import lax
from jax.experimental import pallas as pl
from jax.experimental.pallas import tpu as pltpu
from jax.experimental.pallas import tpu_sc as plsc

_SCALE = 0.0625
_RS = 7
_SR = 2
_PD = 10
_NBINS = _RS * _RS
_H = 64
_W = 64
_NIMG = 4
_R = 512
_L = 16
_NW = 32
_RPW = _R // _NW
_TROWS = _NIMG * _NBINS * _H * _W
_NPAIR = _NIMG * _NBINS
_NTASK = 2 * _NPAIR
_YH = _H // 2


def _sc_relayout(bd_flat):
  mesh = plsc.VectorSubcoreMesh(core_axis_name="c", subcore_axis_name="s")
  chunk = _YH * _W

  @functools.partial(
      pl.kernel,
      out_type=jax.ShapeDtypeStruct((_TROWS, _L), jnp.float32),
      mesh=mesh,
      scratch_types=[
          pltpu.VMEM((2, _PD, chunk), jnp.float32),
          pltpu.VMEM((2, chunk, _L), jnp.float32),
          pltpu.SemaphoreType.DMA,
          pltpu.SemaphoreType.DMA,
          pltpu.SemaphoreType.DMA,
          pltpu.SemaphoreType.DMA,
      ],
      compiler_params=pltpu.CompilerParams(use_tc_tiling_on_sc=False, needs_layout_passes=False),
  )
  def k(bd_hbm, tab_hbm, inb, outb, si0, si1, so0, so1):
    wid = lax.axis_index("s") * 2 + lax.axis_index("c")
    iotav = lax.iota(jnp.int32, _L)
    sin = (si0, si1)
    sout = (so0, so1)

    def in_copies(task, slot):
      pair = task // 2
      yh = task - pair * 2
      b = pair // _NBINS
      bin_ = pair - b * _NBINS
      cs = []
      for d in range(_PD):
        off = ((b * (_PD * _NBINS) + d * _NBINS + bin_) * _H + yh * _YH) * _W
        cs.append(pltpu.make_async_copy(
            bd_hbm.at[pl.ds(off, chunk)], inb.at[slot, d], sin[slot]))
      return cs

    def out_copy(task, slot):
      return pltpu.make_async_copy(
          outb.at[slot], tab_hbm.at[pl.ds(task * chunk, chunk)], sout[slot])

    def scatter(slot):
      def row(yy, carry):
        rbase = yy * _W + iotav
        ridx = [rbase + xb * _L for xb in range(_W // _L)]
        for d0 in range(0, _PD, 2):
          vs = [inb[slot, d, pl.ds(yy * _W + xb * _L, _L)]
                for d in (d0, d0 + 1) for xb in range(_W // _L)]
          for j, d in enumerate((d0, d0 + 1)):
            cidx = jnp.full((_L,), d, jnp.int32)
            for xb in range(_W // _L):
              plsc.store_scatter(outb.at[slot], (ridx[xb], cidx),
                                 vs[j * (_W // _L) + xb])
        return carry

      lax.fori_loop(0, _YH, row, 0)

    nfull = _NTASK // _NW
    rest = _NTASK - nfull * _NW

    def task_of(kk):
      return kk * _NW + wid

    for c in in_copies(task_of(0), 0):
      c.start()
    for kk in range(nfull + 1):
      par = kk % 2
      last = kk == nfull
      if not last:
        if kk + 1 < nfull:
          for c in in_copies(task_of(kk + 1), 1 - par):
            c.start()
        else:
          @pl.when(wid < rest)
          def _():
            for c in in_copies(task_of(kk + 1), 1 - par):
              c.start()
        for c in in_copies(task_of(kk), par):
          c.wait()
        if kk >= 2:
          out_copy(task_of(kk - 2), par).wait()
        scatter(par)
        out_copy(task_of(kk), par).start()
      else:
        @pl.when(wid < rest)
        def _():
          for c in in_copies(task_of(kk), par):
            c.wait()
          out_copy(task_of(kk - 2), par).wait()
          scatter(par)
          out_copy(task_of(kk), par).start()

    out_copy(task_of(nfull - 1), (nfull - 1) % 2).wait()

    @pl.when(wid < rest)
    def _():
      out_copy(task_of(nfull), nfull % 2).wait()

    @pl.when(wid >= rest)
    def _():
      out_copy(task_of(nfull - 2), nfull % 2).wait()

  return k(bd_flat)


def _sc_gather(table, rois_flat):
  mesh = plsc.VectorSubcoreMesh(core_axis_name="c", subcore_axis_name="s")

  @functools.partial(
      pl.kernel,
      out_type=jax.ShapeDtypeStruct((_R * _PD * _NBINS,), jnp.float32),
      mesh=mesh,
      scratch_types=[
          pltpu.VMEM((5 * _L,), jnp.float32),
          pltpu.VMEM((2 * _RS, _L), jnp.int32),
          pltpu.VMEM((2 * _RS, _L), jnp.float32),
          pltpu.VMEM((2 * _RS, _L), jnp.float32),
          pltpu.VMEM((2 * _RS, _L), jnp.int32),
          pltpu.VMEM((2 * _RS, _L), jnp.float32),
          pltpu.VMEM((2 * _RS, _L), jnp.float32),
          pltpu.VMEM((2 * _NBINS, 128), jnp.int32),
          pltpu.VMEM((2, 16 * _RPW, _L), jnp.float32),
          pltpu.VMEM((_RPW * _PD * _NBINS,), jnp.float32),
          pltpu.SemaphoreType.DMA,
          pltpu.SemaphoreType.DMA,
      ],
      compiler_params=pltpu.CompilerParams(use_tc_tiling_on_sc=False, needs_layout_passes=False),
  )
  def k(t_hbm, rois_hbm, out_hbm, rois_v, y0b, wy0b, wy1b, x0b, wx0b, wx1b,
        idxb, rowsb, outw, sem0, sem1):
    wid = lax.axis_index("s") * 2 + lax.axis_index("c")
    pltpu.sync_copy(rois_hbm.at[pl.ds(wid * (5 * _L), 5 * _L)], rois_v)
    iotav = lax.iota(jnp.int32, _L)
    dmask = iotav < _PD

    rowbase = rois_v[pl.ds(0, _L)].astype(jnp.int32) * (_NBINS * _H * _W)
    sw = rois_v[pl.ds(1 * _L, _L)] * _SCALE
    sh = rois_v[pl.ds(2 * _L, _L)] * _SCALE
    ew = rois_v[pl.ds(3 * _L, _L)] * _SCALE
    eh = rois_v[pl.ds(4 * _L, _L)] * _SCALE
    bin_w = jnp.maximum(ew - sw, 0.1) / float(_RS)
    bin_h = jnp.maximum(eh - sh, 0.1) / float(_RS)

    def prep(start, binsz, size, lob, w0b, w1b):
      for p in range(_RS):
        for i in range(_SR):
          g = (i + 0.5) / _SR
          t = start + (p + g) * binsz
          mf = jnp.where((t >= -1.0) & (t <= float(size)), 1.0, 0.0)
          tc = jnp.clip(t, 0.0, float(size - 1))
          lo = jnp.minimum(tc.astype(jnp.int32), size - 2)
          fr = tc - lo.astype(jnp.float32)
          lob[p * _SR + i] = lo
          w0b[p * _SR + i] = (1.0 - fr) * mf
          w1b[p * _SR + i] = fr * mf

    prep(sh, bin_h, _H, y0b, wy0b, wy1b)
    prep(sw, bin_w, _W, x0b, wx0b, wx1b)

    samples = ((0, 0), (0, 1), (1, 0), (1, 1))

    def build(bin_, carry):
      ph = bin_ // _RS
      pw = bin_ - ph * _RS
      rb = rowbase + bin_ * (_H * _W)
      y0a = y0b[ph * 2]
      y0c = y0b[ph * 2 + 1]
      x0a = x0b[pw * 2]
      x0c = x0b[pw * 2 + 1]
      yrow = ((y0a * _W, (y0a + 1) * _W), (y0c * _W, (y0c + 1) * _W))
      xcol = ((x0a, x0a + 1), (x0c, x0c + 1))
      for s, (iy, ix) in enumerate(samples):
        for c, (cy, cx) in enumerate(samples):
          sc = s * 4 + c
          idx = rb + yrow[iy][cy] + xcol[ix][cx]
          idxb[2 * bin_ + sc // 8, pl.ds((sc % 8) * _L, _L)] = idx
      return carry

    lax.fori_loop(0, _NBINS, build, 0)

    sems = (sem0, sem1)

    def copies(bin_, slot):
      c0 = pltpu.make_async_copy(
          t_hbm.at[idxb.at[2 * bin_]],
          rowsb.at[slot, pl.ds(0, 8 * _L)], sems[slot])
      c1 = pltpu.make_async_copy(
          t_hbm.at[idxb.at[2 * bin_ + 1]],
          rowsb.at[slot, pl.ds(8 * _L, 8 * _L)], sems[slot])
      return c0, c1

    def compute(bin_, slot):
      ph = bin_ // _RS
      pw = bin_ - ph * _RS
      ry = ph * 2
      rx = pw * 2
      wyr = ((wy0b[ry], wy1b[ry]), (wy0b[ry + 1], wy1b[ry + 1]))
      wxr = ((wx0b[rx], wx1b[rx]), (wx0b[rx + 1], wx1b[rx + 1]))
      obase = iotav * _NBINS + bin_
      for i in range(_RPW):
        v = None
        for s, (iy, ix) in enumerate(samples):
          g0 = rowsb[slot, (s * 4 + 0) * _L + i]
          g1 = rowsb[slot, (s * 4 + 1) * _L + i]
          g2 = rowsb[slot, (s * 4 + 2) * _L + i]
          g3 = rowsb[slot, (s * 4 + 3) * _L + i]
          a = wxr[ix][0][i] * g0 + wxr[ix][1][i] * g1
          b = wxr[ix][0][i] * g2 + wxr[ix][1][i] * g3
          vs = wyr[iy][0][i] * a + wyr[iy][1][i] * b
          v = vs if v is None else jnp.maximum(v, vs)
        plsc.store_scatter(outw, (obase + i * (_PD * _NBINS),), v, mask=dmask)

    first0, first1 = copies(0, 0)
    first0.start()
    first1.start()

    def pair(p, carry):
      for par in range(2):
        bin_ = 2 * p + par
        nxt = bin_ + 1

        @pl.when(nxt < _NBINS)
        def _():
          n0, n1 = copies(nxt, 1 - par)
          n0.start()
          n1.start()

        @pl.when(bin_ < _NBINS)
        def _():
          w0, w1 = copies(bin_, par)
          w0.wait()
          w1.wait()
          compute(bin_, par)
      return carry

    lax.fori_loop(0, (_NBINS + 1) // 2, pair, 0)
    pltpu.sync_copy(
        outw, out_hbm.at[pl.ds(wid * (_RPW * _PD * _NBINS),
                               _RPW * _PD * _NBINS)])

  return k(table, rois_flat)


def kernel(bottom_data, bottom_rois):
  table = _sc_relayout(bottom_data.reshape(_NIMG * _PD * _NBINS * _H * _W))
  rois_flat = (bottom_rois.reshape(_NW, _RPW, 5)
               .transpose(0, 2, 1).reshape(_NW * 5 * _RPW))
  out = _sc_gather(table, rois_flat)
  return out.reshape(_R, _PD, _RS, _RS)
